# baseline (speedup 1.0000x reference)
import functools

import jax
import jax.numpy as jnp
from jax import lax
from jax.experimental import pallas as pl
from jax.experimental.pallas import tpu as pltpu
from jax.experimental.pallas import tpu_sc as plsc

NC = 2
NS = 16
NW = NC * NS
CH = 128
K = 2


def _sc_segment_sum(n_feat_pad, src_rows, dst_rows, zeros):
    NA, D = n_feat_pad.shape
    N = zeros.shape[0]
    RW = src_rows.shape[0] // (NW * CH)
    assert src_rows.shape[0] == NW * RW * CH and RW % (2 * K) == 0
    rpt = (N // NS) & ~7
    rlast = N - rpt * (NS - 1)
    assert rlast % 8 == 0 and rlast > 0

    mesh = plsc.VectorSubcoreMesh(
        core_axis_name="c", subcore_axis_name="s", num_cores=NC, num_subcores=NS)

    @functools.partial(
        pl.kernel,
        out_type=jax.ShapeDtypeStruct((NC, N, D), jnp.float32),
        mesh=mesh,
        scratch_types=[
            *[pltpu.VMEM((K, CH), jnp.int32) for _ in range(2)],
            *[pltpu.VMEM((K, CH), jnp.int32) for _ in range(2)],
            *[pltpu.VMEM((CH, D), jnp.float32) for _ in range(K)],
            pltpu.VMEM_SHARED((NA, D), jnp.float32),
            pltpu.SemaphoreType.DMA,
            pltpu.SemaphoreType.DMA,
            pltpu.SemaphoreType.DMA,
            pltpu.SemaphoreType.DMA,
        ],
    )
    def seg_sum(nf_hbm, src_hbm, dst_hbm, z_hbm, out_hbm, *rest):
        sidx = rest[0:2]
        didx = rest[2:4]
        rows = rest[4:4 + K]
        acc, gsem, ssem = rest[4 + K], rest[5 + K], rest[6 + K]
        isem = rest[7 + K:9 + K]
        c = lax.axis_index("c")
        s = lax.axis_index("s")
        wid = s * NC + c
        r0 = pl.multiple_of(s * rpt, 8)

        @pl.when(s < NS - 1)
        def _():
            pltpu.sync_copy(z_hbm.at[pl.ds(r0, rpt)], acc.at[pl.ds(r0, rpt)])

        @pl.when(s == NS - 1)
        def _():
            pltpu.sync_copy(z_hbm.at[pl.ds(r0, rlast)], acc.at[pl.ds(r0, rlast)])

        ib = wid * (RW * CH)
        NG = RW // K

        def fire_idx(g, p):
            ds = []
            for b in range(K):
                off = pl.multiple_of(ib + (g * K + b) * CH, 8)
                ds.append(pltpu.async_copy(
                    src_hbm.at[pl.ds(off, CH)], sidx[p].at[b], isem[p]))
                ds.append(pltpu.async_copy(
                    dst_hbm.at[pl.ds(off, CH)], didx[p].at[b], isem[p]))
            return ds

        def do_group(g, p, last):
            nxt = [] if last else fire_idx(g + 1, 1 - p)
            gd = [pltpu.async_copy(nf_hbm.at[sidx[p].at[b]], rows[b], gsem)
                  for b in range(K)]
            for d in gd:
                d.wait()
            sd = [pltpu.async_copy(rows[b], acc.at[didx[p].at[b]], ssem,
                                   add=True)
                  for b in range(K)]
            for d in sd:
                d.wait()
            for d in nxt:
                d.wait()

        plsc.subcore_barrier()

        for d in fire_idx(0, 0):
            d.wait()

        def two_groups(i, _):
            do_group(i * 2, 0, False)
            do_group(i * 2 + 1, 1, False)
            return 0

        lax.fori_loop(0, (NG - 2) // 2, two_groups, 0)
        do_group(NG - 2, 0, False)
        do_group(NG - 1, 1, True)

        plsc.subcore_barrier()

        @pl.when(s < NS - 1)
        def _():
            pltpu.sync_copy(acc.at[pl.ds(r0, rpt)], out_hbm.at[c, pl.ds(r0, rpt)])

        @pl.when(s == NS - 1)
        def _():
            pltpu.sync_copy(acc.at[pl.ds(r0, rlast)],
                            out_hbm.at[c, pl.ds(r0, rlast)])

    return seg_sum(n_feat_pad, src_rows, dst_rows, zeros)


def _tc_mlp(partials, W1, b1, W2, b2, W3, b3, W4, b4):
    _, N, D = partials.shape
    H = W1.shape[1]
    NB = W2.shape[1]
    BN = 1000
    assert N % BN == 0
    grid = N // BN

    def body(p_ref, W1_ref, b1_ref, W2_ref, b2_ref, W3_ref, b3_ref,
             W4_ref, b4_ref, fps_ref, out_ref, acc_ref):
        i = pl.program_id(0)
        h = p_ref[0] + p_ref[1]
        r = jnp.maximum(
            jnp.dot(h, W1_ref[...], preferred_element_type=jnp.float32)
            + b1_ref[...], 0.0)
        lg = (jnp.dot(r, W2_ref[...], preferred_element_type=jnp.float32)
              + b2_ref[...])
        m = jnp.max(lg, axis=1, keepdims=True)
        e = jnp.exp(lg - m)
        p = e / jnp.sum(e, axis=1, keepdims=True)
        colsum = jnp.sum(p, axis=0, keepdims=True)

        @pl.when(i == 0)
        def _():
            acc_ref[...] = colsum

        @pl.when(i > 0)
        def _():
            acc_ref[...] += colsum

        @pl.when(i == pl.num_programs(0) - 1)
        def _():
            fps = acc_ref[...] * 2.0
            fps_ref[...] = fps
            o = jnp.maximum(
                jnp.dot(fps, W3_ref[...], preferred_element_type=jnp.float32)
                + b3_ref[...], 0.0)
            out_ref[...] = (
                jnp.dot(o, W4_ref[...], preferred_element_type=jnp.float32)
                + b4_ref[...])

    fixed = lambda *_: (0, 0)
    return pl.pallas_call(
        body,
        grid=(grid,),
        in_specs=[
            pl.BlockSpec((2, BN, D), lambda i: (0, i, 0)),
            pl.BlockSpec((D, H), fixed),
            pl.BlockSpec((1, H), fixed),
            pl.BlockSpec((H, NB), fixed),
            pl.BlockSpec((1, NB), fixed),
            pl.BlockSpec((NB, H), fixed),
            pl.BlockSpec((1, H), fixed),
            pl.BlockSpec((H, 1), fixed),
            pl.BlockSpec((1, 1), fixed),
        ],
        out_specs=[
            pl.BlockSpec((1, NB), fixed),
            pl.BlockSpec((1, 1), fixed),
        ],
        out_shape=[
            jax.ShapeDtypeStruct((1, NB), jnp.float32),
            jax.ShapeDtypeStruct((1, 1), jnp.float32),
        ],
        scratch_shapes=[pltpu.VMEM((1, NB), jnp.float32)],
    )(partials, W1, b1.reshape(1, H), W2, b2.reshape(1, NB),
      W3, b3.reshape(1, H), W4, b4.reshape(1, 1))


def kernel(n_feat, edge_index, W1, b1, W2, b2, W3, b3, W4, b4):
    N, D = n_feat.shape
    E = edge_index.shape[1]
    rw = -(-E // (NW * CH * 2 * K)) * 2 * K
    e_pad = NW * rw * CH
    pad = e_pad - E
    src = jnp.concatenate([edge_index[0], jnp.full((pad,), N, jnp.int32)])
    dst = jnp.concatenate([edge_index[1], jnp.full((pad,), N, jnp.int32)])
    n_feat_pad = jnp.pad(n_feat, ((0, 8), (0, 0)))
    zeros = jnp.zeros((N, D), dtype=jnp.float32)
    partials = _sc_segment_sum(n_feat_pad, src, dst, zeros)
    fps, out = _tc_mlp(partials, W1, b1, W2, b2, W3, b3, W4, b4)
    return (fps, out.squeeze(0))


# speedup vs baseline: 1.0817x; 1.0817x over previous
import functools

import jax
import jax.numpy as jnp
from jax import lax
from jax.experimental import pallas as pl
from jax.experimental.pallas import tpu as pltpu
from jax.experimental.pallas import tpu_sc as plsc

NC = 2
NS = 16
NW = NC * NS
CH = 128
K = 2


def _sc_segment_sum(n_feat_pad, edge_rows, zeros):
    NA, D = n_feat_pad.shape
    N = zeros.shape[0]
    RW = edge_rows.shape[0] // NW
    assert edge_rows.shape[0] == NW * RW and RW % 2 == 0 and RW >= 4
    rpt = (N // NS) & ~7
    rlast = N - rpt * (NS - 1)
    assert rlast % 8 == 0 and rlast > 0

    mesh = plsc.VectorSubcoreMesh(
        core_axis_name="c", subcore_axis_name="s", num_cores=NC, num_subcores=NS)

    @functools.partial(
        pl.kernel,
        out_type=jax.ShapeDtypeStruct((NC, N, D), jnp.float32),
        mesh=mesh,
        scratch_types=[
            *[pltpu.VMEM((2, CH), jnp.int32) for _ in range(2)],
            *[pltpu.VMEM((CH, D), jnp.float32) for _ in range(K)],
            pltpu.VMEM_SHARED((NA, D), jnp.float32),
            pltpu.SemaphoreType.DMA,
            pltpu.SemaphoreType.DMA,
        ],
    )
    def seg_sum(nf_hbm, e_hbm, z_hbm, out_hbm, *rest):
        eidx = rest[0:2]
        rows = rest[2:2 + K]
        acc = rest[2 + K]
        gsem = rest[3 + K:5 + K]
        c = lax.axis_index("c")
        s = lax.axis_index("s")
        wid = s * NC + c
        r0 = pl.multiple_of(s * rpt, 8)

        @pl.when(s < NS - 1)
        def _():
            pltpu.sync_copy(z_hbm.at[pl.ds(r0, rpt)], acc.at[pl.ds(r0, rpt)])

        @pl.when(s == NS - 1)
        def _():
            pltpu.sync_copy(z_hbm.at[pl.ds(r0, rlast)], acc.at[pl.ds(r0, rlast)])

        wr = wid * RW

        def fire_gather(j, p):
            return pltpu.async_copy(
                nf_hbm.at[eidx[p].at[0]], rows[p], gsem[p])

        def wait_gather(p):
            pltpu.make_async_copy(nf_hbm.at[eidx[p].at[0]], rows[p],
                                  gsem[p]).wait()

        def step(j, p, prefetch):
            if prefetch:
                pltpu.sync_copy(e_hbm.at[j + 1], eidx[1 - p])
                fire_gather(j + 1, 1 - p)
            wait_gather(p)
            pltpu.sync_copy(rows[p], acc.at[eidx[p].at[1]], add=True)

        plsc.subcore_barrier()

        pltpu.sync_copy(e_hbm.at[wr], eidx[0])
        fire_gather(wr, 0)

        def two_steps(g, _):
            step(wr + 2 * g, 0, True)
            step(wr + 2 * g + 1, 1, True)
            return 0

        lax.fori_loop(0, (RW - 2) // 2, two_steps, 0)
        step(wr + RW - 2, 0, True)
        step(wr + RW - 1, 1, False)

        plsc.subcore_barrier()

        @pl.when(s < NS - 1)
        def _():
            pltpu.sync_copy(acc.at[pl.ds(r0, rpt)], out_hbm.at[c, pl.ds(r0, rpt)])

        @pl.when(s == NS - 1)
        def _():
            pltpu.sync_copy(acc.at[pl.ds(r0, rlast)],
                            out_hbm.at[c, pl.ds(r0, rlast)])

    return seg_sum(n_feat_pad, edge_rows, zeros)


def _tc_mlp(partials, W1, b1, W2, b2, W3, b3, W4, b4):
    _, N, D = partials.shape
    H = W1.shape[1]
    NB = W2.shape[1]
    BN = 1000
    assert N % BN == 0
    grid = N // BN

    def body(p_ref, W1_ref, b1_ref, W2_ref, b2_ref, W3_ref, b3_ref,
             W4_ref, b4_ref, fps_ref, out_ref, acc_ref):
        i = pl.program_id(0)
        h = p_ref[0] + p_ref[1]
        r = jnp.maximum(
            jnp.dot(h, W1_ref[...], preferred_element_type=jnp.float32)
            + b1_ref[...], 0.0)
        lg = (jnp.dot(r, W2_ref[...], preferred_element_type=jnp.float32)
              + b2_ref[...])
        m = jnp.max(lg, axis=1, keepdims=True)
        e = jnp.exp(lg - m)
        p = e / jnp.sum(e, axis=1, keepdims=True)
        colsum = jnp.sum(p, axis=0, keepdims=True)

        @pl.when(i == 0)
        def _():
            acc_ref[...] = colsum

        @pl.when(i > 0)
        def _():
            acc_ref[...] += colsum

        @pl.when(i == pl.num_programs(0) - 1)
        def _():
            fps = acc_ref[...] * 2.0
            fps_ref[...] = fps
            o = jnp.maximum(
                jnp.dot(fps, W3_ref[...], preferred_element_type=jnp.float32)
                + b3_ref[...], 0.0)
            out_ref[...] = (
                jnp.dot(o, W4_ref[...], preferred_element_type=jnp.float32)
                + b4_ref[...])

    fixed = lambda *_: (0, 0)
    return pl.pallas_call(
        body,
        grid=(grid,),
        in_specs=[
            pl.BlockSpec((2, BN, D), lambda i: (0, i, 0)),
            pl.BlockSpec((D, H), fixed),
            pl.BlockSpec((1, H), fixed),
            pl.BlockSpec((H, NB), fixed),
            pl.BlockSpec((1, NB), fixed),
            pl.BlockSpec((NB, H), fixed),
            pl.BlockSpec((1, H), fixed),
            pl.BlockSpec((H, 1), fixed),
            pl.BlockSpec((1, 1), fixed),
        ],
        out_specs=[
            pl.BlockSpec((1, NB), fixed),
            pl.BlockSpec((1, 1), fixed),
        ],
        out_shape=[
            jax.ShapeDtypeStruct((1, NB), jnp.float32),
            jax.ShapeDtypeStruct((1, 1), jnp.float32),
        ],
        scratch_shapes=[pltpu.VMEM((1, NB), jnp.float32)],
    )(partials, W1, b1.reshape(1, H), W2, b2.reshape(1, NB),
      W3, b3.reshape(1, H), W4, b4.reshape(1, 1))


def kernel(n_feat, edge_index, W1, b1, W2, b2, W3, b3, W4, b4):
    N, D = n_feat.shape
    E = edge_index.shape[1]
    rw = -(-E // (NW * CH * 4)) * 4
    e_pad = NW * rw * CH
    pad = e_pad - E
    src = jnp.concatenate([edge_index[0], jnp.full((pad,), N, jnp.int32)])
    dst = jnp.concatenate([edge_index[1], jnp.full((pad,), N, jnp.int32)])
    edge_rows = jnp.stack(
        [src.reshape(-1, CH), dst.reshape(-1, CH)], axis=1)
    n_feat_pad = jnp.pad(n_feat, ((0, 8), (0, 0)))
    zeros = jnp.zeros((N, D), dtype=jnp.float32)
    partials = _sc_segment_sum(n_feat_pad, edge_rows, zeros)
    fps, out = _tc_mlp(partials, W1, b1, W2, b2, W3, b3, W4, b4)
    return (fps, out.squeeze(0))


# speedup vs baseline: 2.7615x; 2.5530x over previous
import functools

import jax
import jax.numpy as jnp
from jax import lax
from jax.experimental import pallas as pl
from jax.experimental.pallas import tpu as pltpu
from jax.experimental.pallas import tpu_sc as plsc

NC = 2
NS = 16
NW = NC * NS
CH = 128
K = 2


def _sc_segment_sum(n_feat_pad, edge_rows, zeros):
    NA, D = n_feat_pad.shape
    N = zeros.shape[0]
    RW = edge_rows.shape[0] // NW
    assert edge_rows.shape[0] == NW * RW and RW % 2 == 0 and RW >= 4
    rpt = (N // NS) & ~7
    rlast = N - rpt * (NS - 1)
    assert rlast % 8 == 0 and rlast > 0

    mesh = plsc.VectorSubcoreMesh(
        core_axis_name="c", subcore_axis_name="s", num_cores=NC, num_subcores=NS)

    @functools.partial(
        pl.kernel,
        out_type=jax.ShapeDtypeStruct((NC, N, D), jnp.float32),
        mesh=mesh,
        scratch_types=[
            *[pltpu.VMEM((2, CH), jnp.int32) for _ in range(2)],
            *[pltpu.VMEM((CH, D), jnp.float32) for _ in range(K)],
            pltpu.VMEM_SHARED((N, D), jnp.float32),
            pltpu.SemaphoreType.DMA,
            pltpu.SemaphoreType.DMA,
        ],
    )
    def seg_sum(nf_hbm, e_hbm, z_hbm, out_hbm, *rest):
        eidx = rest[0:2]
        rows = rest[2:2 + K]
        acc = rest[2 + K]
        gsem = rest[3 + K:5 + K]
        c = lax.axis_index("c")
        s = lax.axis_index("s")
        wid = s * NC + c
        r0 = pl.multiple_of(s * rpt, 8)

        @pl.when(s < NS - 1)
        def _():
            pltpu.sync_copy(z_hbm.at[pl.ds(r0, rpt)], acc.at[pl.ds(r0, rpt)])

        @pl.when(s == NS - 1)
        def _():
            pltpu.sync_copy(z_hbm.at[pl.ds(r0, rlast)], acc.at[pl.ds(r0, rlast)])

        wr = wid * RW

        def fire_gather(j, p):
            return pltpu.async_copy(
                nf_hbm.at[eidx[p].at[0]], rows[p], gsem[p])

        def wait_gather(p):
            pltpu.make_async_copy(nf_hbm.at[eidx[p].at[0]], rows[p],
                                  gsem[p]).wait()

        def step(j, p, prefetch):
            if prefetch:
                pltpu.sync_copy(e_hbm.at[j + 1], eidx[1 - p])
                fire_gather(j + 1, 1 - p)
            wait_gather(p)
            pltpu.sync_copy(rows[p], acc.at[eidx[p].at[1]], add=True)

        plsc.subcore_barrier()

        pltpu.sync_copy(e_hbm.at[wr], eidx[0])
        fire_gather(wr, 0)

        def two_steps(g, _):
            step(wr + 2 * g, 0, True)
            step(wr + 2 * g + 1, 1, True)
            return 0

        lax.fori_loop(0, (RW - 2) // 2, two_steps, 0)
        step(wr + RW - 2, 0, True)
        step(wr + RW - 1, 1, False)

        plsc.subcore_barrier()

        @pl.when(s < NS - 1)
        def _():
            pltpu.sync_copy(acc.at[pl.ds(r0, rpt)], out_hbm.at[c, pl.ds(r0, rpt)])

        @pl.when(s == NS - 1)
        def _():
            pltpu.sync_copy(acc.at[pl.ds(r0, rlast)],
                            out_hbm.at[c, pl.ds(r0, rlast)])

    return seg_sum(n_feat_pad, edge_rows, zeros)


def _tc_mlp(partials, W1, b1, W2, b2, W3, b3, W4, b4):
    _, N, D = partials.shape
    H = W1.shape[1]
    NB = W2.shape[1]
    BN = 1000
    assert N % BN == 0
    grid = N // BN

    def body(p_ref, W1_ref, b1_ref, W2_ref, b2_ref, W3_ref, b3_ref,
             W4_ref, b4_ref, fps_ref, out_ref, acc_ref):
        i = pl.program_id(0)
        h = p_ref[0] + p_ref[1]
        r = jnp.maximum(
            jnp.dot(h, W1_ref[...], preferred_element_type=jnp.float32)
            + b1_ref[...], 0.0)
        lg = (jnp.dot(r, W2_ref[...], preferred_element_type=jnp.float32)
              + b2_ref[...])
        m = jnp.max(lg, axis=1, keepdims=True)
        e = jnp.exp(lg - m)
        p = e / jnp.sum(e, axis=1, keepdims=True)
        colsum = jnp.sum(p, axis=0, keepdims=True)

        @pl.when(i == 0)
        def _():
            acc_ref[...] = colsum

        @pl.when(i > 0)
        def _():
            acc_ref[...] += colsum

        @pl.when(i == pl.num_programs(0) - 1)
        def _():
            fps = acc_ref[...] * 2.0
            fps_ref[...] = fps
            o = jnp.maximum(
                jnp.dot(fps, W3_ref[...], preferred_element_type=jnp.float32)
                + b3_ref[...], 0.0)
            out_ref[...] = (
                jnp.dot(o, W4_ref[...], preferred_element_type=jnp.float32)
                + b4_ref[...])

    fixed = lambda *_: (0, 0)
    return pl.pallas_call(
        body,
        grid=(grid,),
        in_specs=[
            pl.BlockSpec((2, BN, D), lambda i: (0, i, 0)),
            pl.BlockSpec((D, H), fixed),
            pl.BlockSpec((1, H), fixed),
            pl.BlockSpec((H, NB), fixed),
            pl.BlockSpec((1, NB), fixed),
            pl.BlockSpec((NB, H), fixed),
            pl.BlockSpec((1, H), fixed),
            pl.BlockSpec((H, 1), fixed),
            pl.BlockSpec((1, 1), fixed),
        ],
        out_specs=[
            pl.BlockSpec((1, NB), fixed),
            pl.BlockSpec((1, 1), fixed),
        ],
        out_shape=[
            jax.ShapeDtypeStruct((1, NB), jnp.float32),
            jax.ShapeDtypeStruct((1, 1), jnp.float32),
        ],
        scratch_shapes=[pltpu.VMEM((1, NB), jnp.float32)],
    )(partials, W1, b1.reshape(1, H), W2, b2.reshape(1, NB),
      W3, b3.reshape(1, H), W4, b4.reshape(1, 1))


def kernel(n_feat, edge_index, W1, b1, W2, b2, W3, b3, W4, b4):
    N, D = n_feat.shape
    E = edge_index.shape[1]
    rw = -(-E // (NW * CH * 4)) * 4
    e_pad = NW * rw * CH
    pad = e_pad - E
    ar = jnp.arange(pad, dtype=jnp.int32)
    src = jnp.concatenate([edge_index[0], N + (ar % 8)])
    dst = jnp.concatenate([edge_index[1], ar % N])
    edge_rows = jnp.stack(
        [src.reshape(-1, CH), dst.reshape(-1, CH)], axis=1)
    n_feat_pad = jnp.pad(n_feat, ((0, 8), (0, 0)))
    zeros = jnp.zeros((N, D), dtype=jnp.float32)
    partials = _sc_segment_sum(n_feat_pad, edge_rows, zeros)
    fps, out = _tc_mlp(partials, W1, b1, W2, b2, W3, b3, W4, b4)
    return (fps, out.squeeze(0))


# speedup vs baseline: 3.0033x; 1.0876x over previous
import functools

import jax
import jax.numpy as jnp
from jax import lax
from jax.experimental import pallas as pl
from jax.experimental.pallas import tpu as pltpu
from jax.experimental.pallas import tpu_sc as plsc

NC = 2
NS = 16
NW = NC * NS
CH = 128
K = 2


JUNK = 128


def _sc_segment_sum(n_feat, edge_rows, zeros):
    N, D = n_feat.shape
    RW = edge_rows.shape[0] // NW
    assert edge_rows.shape[0] == NW * RW and RW % 2 == 0 and RW >= 4
    rpt = (N // NS) & ~7
    rlast = N - rpt * (NS - 1)
    assert rlast % 8 == 0 and rlast > 0

    mesh = plsc.VectorSubcoreMesh(
        core_axis_name="c", subcore_axis_name="s", num_cores=NC, num_subcores=NS)

    @functools.partial(
        pl.kernel,
        out_type=jax.ShapeDtypeStruct((NC, N, D), jnp.float32),
        mesh=mesh,
        scratch_types=[
            *[pltpu.VMEM((2, CH), jnp.int32) for _ in range(2)],
            *[pltpu.VMEM((CH, D), jnp.float32) for _ in range(K)],
            pltpu.VMEM_SHARED((N + JUNK, D), jnp.float32),
            pltpu.SemaphoreType.DMA,
            pltpu.SemaphoreType.DMA,
        ],
    )
    def seg_sum(nf_hbm, e_hbm, z_hbm, out_hbm, *rest):
        eidx = rest[0:2]
        rows = rest[2:2 + K]
        acc = rest[2 + K]
        gsem = rest[3 + K:5 + K]
        c = lax.axis_index("c")
        s = lax.axis_index("s")
        wid = s * NC + c
        r0 = pl.multiple_of(s * rpt, 8)

        @pl.when(s < NS - 1)
        def _():
            pltpu.sync_copy(z_hbm.at[pl.ds(0, rpt)], acc.at[pl.ds(r0, rpt)])

        @pl.when(s == NS - 1)
        def _():
            pltpu.sync_copy(z_hbm.at[pl.ds(0, rlast)], acc.at[pl.ds(r0, rlast)])

        wr = wid * RW

        def fire_gather(j, p):
            return pltpu.async_copy(
                nf_hbm.at[eidx[p].at[0]], rows[p], gsem[p])

        def wait_gather(p):
            pltpu.make_async_copy(nf_hbm.at[eidx[p].at[0]], rows[p],
                                  gsem[p]).wait()

        def step(j, p, prefetch):
            if prefetch:
                pltpu.sync_copy(e_hbm.at[j + 1], eidx[1 - p])
                fire_gather(j + 1, 1 - p)
            wait_gather(p)
            pltpu.sync_copy(rows[p], acc.at[eidx[p].at[1]], add=True)

        plsc.subcore_barrier()

        pltpu.sync_copy(e_hbm.at[wr], eidx[0])
        fire_gather(wr, 0)

        def two_steps(g, _):
            step(wr + 2 * g, 0, True)
            step(wr + 2 * g + 1, 1, True)
            return 0

        lax.fori_loop(0, (RW - 2) // 2, two_steps, 0)
        step(wr + RW - 2, 0, True)
        step(wr + RW - 1, 1, False)

        plsc.subcore_barrier()

        @pl.when(s < NS - 1)
        def _():
            pltpu.sync_copy(acc.at[pl.ds(r0, rpt)], out_hbm.at[c, pl.ds(r0, rpt)])

        @pl.when(s == NS - 1)
        def _():
            pltpu.sync_copy(acc.at[pl.ds(r0, rlast)],
                            out_hbm.at[c, pl.ds(r0, rlast)])

    return seg_sum(n_feat, edge_rows, zeros)


def _tc_mlp(partials, W1, b1, W2, b2, W3, b3, W4, b4):
    _, N, D = partials.shape
    H = W1.shape[1]
    NB = W2.shape[1]
    BN = 1000
    assert N % BN == 0
    grid = N // BN

    def body(p_ref, W1_ref, b1_ref, W2_ref, b2_ref, W3_ref, b3_ref,
             W4_ref, b4_ref, fps_ref, out_ref, acc_ref):
        i = pl.program_id(0)
        h = p_ref[0] + p_ref[1]
        r = jnp.maximum(
            jnp.dot(h, W1_ref[...], preferred_element_type=jnp.float32)
            + b1_ref[...], 0.0)
        lg = (jnp.dot(r, W2_ref[...], preferred_element_type=jnp.float32)
              + b2_ref[...])
        m = jnp.max(lg, axis=1, keepdims=True)
        e = jnp.exp(lg - m)
        p = e / jnp.sum(e, axis=1, keepdims=True)
        colsum = jnp.sum(p, axis=0, keepdims=True)

        @pl.when(i == 0)
        def _():
            acc_ref[...] = colsum

        @pl.when(i > 0)
        def _():
            acc_ref[...] += colsum

        @pl.when(i == pl.num_programs(0) - 1)
        def _():
            fps = acc_ref[...] * 2.0
            fps_ref[...] = fps
            o = jnp.maximum(
                jnp.dot(fps, W3_ref[...], preferred_element_type=jnp.float32)
                + b3_ref[...], 0.0)
            out_ref[...] = (
                jnp.dot(o, W4_ref[...], preferred_element_type=jnp.float32)
                + b4_ref[...])

    fixed = lambda *_: (0, 0)
    return pl.pallas_call(
        body,
        grid=(grid,),
        in_specs=[
            pl.BlockSpec((2, BN, D), lambda i: (0, i, 0)),
            pl.BlockSpec((D, H), fixed),
            pl.BlockSpec((1, H), fixed),
            pl.BlockSpec((H, NB), fixed),
            pl.BlockSpec((1, NB), fixed),
            pl.BlockSpec((NB, H), fixed),
            pl.BlockSpec((1, H), fixed),
            pl.BlockSpec((H, 1), fixed),
            pl.BlockSpec((1, 1), fixed),
        ],
        out_specs=[
            pl.BlockSpec((1, NB), fixed),
            pl.BlockSpec((1, 1), fixed),
        ],
        out_shape=[
            jax.ShapeDtypeStruct((1, NB), jnp.float32),
            jax.ShapeDtypeStruct((1, 1), jnp.float32),
        ],
        scratch_shapes=[pltpu.VMEM((1, NB), jnp.float32)],
    )(partials, W1, b1.reshape(1, H), W2, b2.reshape(1, NB),
      W3, b3.reshape(1, H), W4, b4.reshape(1, 1))


def kernel(n_feat, edge_index, W1, b1, W2, b2, W3, b3, W4, b4):
    N, D = n_feat.shape
    E = edge_index.shape[1]
    rw = -(-E // (NW * CH * 4)) * 4
    e_pad = NW * rw * CH
    pad = e_pad - E
    ar = jnp.arange(pad, dtype=jnp.int32)
    src = jnp.concatenate([edge_index[0], ar % N])
    dst = jnp.concatenate([edge_index[1], N + (ar % JUNK)])
    edge_rows = jnp.stack(
        [src.reshape(-1, CH), dst.reshape(-1, CH)], axis=1)
    rlast = N - ((N // NS) & ~7) * (NS - 1)
    zeros = jnp.zeros((rlast, D), dtype=jnp.float32)
    partials = _sc_segment_sum(n_feat, edge_rows, zeros)
    fps, out = _tc_mlp(partials, W1, b1, W2, b2, W3, b3, W4, b4)
    return (fps, out.squeeze(0))
